# Initial kernel scaffold; baseline (speedup 1.0000x reference)
import functools

import jax
import jax.numpy as jnp
from jax import lax
from jax.experimental import pallas as pl
from jax.experimental.pallas import tpu as pltpu
from jax.experimental.pallas import tpu_sc as plsc

_N = 320000
_D = 128
_E = 256
_NC = 2
_NS = 16
_NW = _NC * _NS
_ROWS_W = _N // _NW
_CH = 80
_NCH = _ROWS_W // _CH

_mesh = plsc.VectorSubcoreMesh(core_axis_name="c", subcore_axis_name="s")


@functools.partial(
    pl.kernel,
    out_type=(
        jax.ShapeDtypeStruct((_NC, 3, _E, _D), jnp.float32),
        jax.ShapeDtypeStruct((_NC, 3, _E, 16), jnp.float32),
    ),
    mesh=_mesh,
    scratch_types=[
        pltpu.VMEM((_CH, _D), jnp.float32),
        pltpu.VMEM((_CH,), jnp.int32),
        pltpu.VMEM((_CH, 16), jnp.float32),
        pltpu.VMEM_SHARED((_E, _D), jnp.float32),
        pltpu.VMEM_SHARED((_E, _D), jnp.float32),
        pltpu.VMEM_SHARED((_E, _D), jnp.float32),
        pltpu.VMEM_SHARED((_E, 16), jnp.float32),
        pltpu.VMEM_SHARED((_E, 16), jnp.float32),
        pltpu.VMEM_SHARED((_E, 16), jnp.float32),
    ],
)
def _segsum_sc(x_u, x_v, x_y, b_u, b_v, b_y, z_pl, z_cnt, ones_in,
               sums_out, cnt_out,
               dbuf, ibuf, ones, acc_u, acc_v, acc_y, cnt_u, cnt_v, cnt_y):
    cid = lax.axis_index("c")
    sid = lax.axis_index("s")
    wid = cid * _NS + sid
    accs = (acc_u, acc_v, acc_y)
    cnts = (cnt_u, cnt_v, cnt_y)

    row0 = pl.multiple_of(sid * 16, 16)
    for p in range(3):
        pltpu.sync_copy(z_pl.at[pl.ds(row0, 16)], accs[p].at[pl.ds(row0, 16)])
        pltpu.sync_copy(z_cnt.at[pl.ds(row0, 16)], cnts[p].at[pl.ds(row0, 16)])
    pltpu.sync_copy(ones_in, ones)
    plsc.subcore_barrier()

    base = wid * _ROWS_W
    for p, (x, b) in enumerate(((x_u, b_u), (x_v, b_v), (x_y, b_y))):
        acc, cnt = accs[p], cnts[p]

        def body(i, carry, x=x, b=b, acc=acc, cnt=cnt):
            off = pl.multiple_of(base + i * _CH, 8)
            pltpu.sync_copy(x.at[pl.ds(off, _CH)], dbuf)
            pltpu.sync_copy(b.at[pl.ds(off, _CH)], ibuf)
            pltpu.sync_copy(dbuf, acc.at[ibuf], add=True)
            pltpu.sync_copy(ones, cnt.at[ibuf], add=True)
            return carry

        lax.fori_loop(0, _NCH, body, 0)

    plsc.subcore_barrier()
    for p in range(3):
        pltpu.sync_copy(accs[p].at[pl.ds(row0, 16)],
                        sums_out.at[cid, p, pl.ds(row0, 16)])
        pltpu.sync_copy(cnts[p].at[pl.ds(row0, 16)],
                        cnt_out.at[cid, p, pl.ds(row0, 16)])


def _mlp_tc(s_ref, c_ref, w1_ref, b1_ref, w2_ref, b2_ref, w3_ref, b3_ref, o_ref):
    s = s_ref[0] + s_ref[1]
    c = c_ref[0] + c_ref[1]
    m = s / jnp.maximum(c, 1.0)
    h = (jnp.dot(m[0:_E], w1_ref[0:_D], preferred_element_type=jnp.float32)
         + jnp.dot(m[_E:2 * _E], w1_ref[_D:2 * _D], preferred_element_type=jnp.float32)
         + jnp.dot(m[2 * _E:3 * _E], w1_ref[2 * _D:3 * _D], preferred_element_type=jnp.float32))
    h = jax.nn.relu(h + b1_ref[0])
    h = jax.nn.relu(jnp.dot(h, w2_ref[...], preferred_element_type=jnp.float32) + b2_ref[0])
    o_ref[...] = jnp.dot(h, w3_ref[...], preferred_element_type=jnp.float32) + b3_ref[0]


def kernel(x_u, x_v, x_y, batch_u, batch_v, batch_y, W1, b1, W2, b2, W3, b3):
    z_pl = jnp.zeros((_E, _D), jnp.float32)
    z_cnt = jnp.zeros((_E, 16), jnp.float32)
    ones_in = jnp.ones((_CH, 16), jnp.float32)
    sums, cnts = _segsum_sc(x_u, x_v, x_y, batch_u, batch_v, batch_y,
                            z_pl, z_cnt, ones_in)

    s2 = sums.reshape(_NC, 3 * _E, _D)
    c2 = cnts[..., :1].reshape(_NC, 3 * _E, 1)
    W3p = jnp.pad(W3, ((0, 0), (0, 128 - W3.shape[1])))
    b3p = jnp.pad(b3, (0, 128 - b3.shape[0])).reshape(1, 128)
    out = pl.pallas_call(
        _mlp_tc,
        out_shape=jax.ShapeDtypeStruct((_E, 128), jnp.float32),
    )(s2, c2, W1, b1.reshape(1, _D), W2, b2.reshape(1, 64), W3p, b3p)
    return out[:, :3]


# speedup vs baseline: 3.6051x; 3.6051x over previous
import functools

import jax
import jax.numpy as jnp
from jax import lax
from jax.experimental import pallas as pl
from jax.experimental.pallas import tpu as pltpu
from jax.experimental.pallas import tpu_sc as plsc

_N = 320000
_D = 128
_E = 256
_NS = 16
_ROWS_W = _N // _NS
_CH = 80
_NCH = _ROWS_W // _CH

_HB = 2048
_NHB = _N // _HB
_NPAD = 157 * _HB

_mesh = plsc.VectorSubcoreMesh(core_axis_name="c", subcore_axis_name="s",
                               num_cores=1)


@functools.partial(
    pl.kernel,
    out_type=jax.ShapeDtypeStruct((3, _E, _D), jnp.float32),
    mesh=_mesh,
    scratch_types=[
        pltpu.VMEM((2, _CH, _D), jnp.float32),
        pltpu.VMEM((2, _CH), jnp.int32),
        pltpu.VMEM_SHARED((_E, _D), jnp.float32),
        pltpu.VMEM_SHARED((_E, _D), jnp.float32),
        pltpu.VMEM_SHARED((_E, _D), jnp.float32),
        pltpu.SemaphoreType.DMA,
        pltpu.SemaphoreType.DMA,
    ],
)
def _segsum_sc(x_u, x_v, x_y, b_u, b_v, b_y, z_pl,
               sums_out,
               dbuf, ibuf, acc_u, acc_v, acc_y, sem0, sem1):
    sid = lax.axis_index("s")
    accs = (acc_u, acc_v, acc_y)

    row0 = pl.multiple_of(sid * 16, 16)
    for p in range(3):
        pltpu.sync_copy(z_pl.at[pl.ds(row0, 16)], accs[p].at[pl.ds(row0, 16)])
    plsc.subcore_barrier()

    base = sid * _ROWS_W
    for p, (x, b) in enumerate(((x_u, b_u), (x_v, b_v), (x_y, b_y))):
        acc = accs[p]

        pltpu.async_copy(x.at[pl.ds(base, _CH)], dbuf.at[0], sem0)
        pltpu.async_copy(b.at[pl.ds(base, _CH)], ibuf.at[0], sem0)

        def body(k, carry, x=x, b=b, acc=acc):
            i0 = 2 * k
            off0 = pl.multiple_of(base + i0 * _CH, 8)
            off1 = pl.multiple_of(base + (i0 + 1) * _CH, 8)
            pltpu.make_async_copy(x.at[pl.ds(off0, _CH)], dbuf.at[0], sem0).wait()
            pltpu.make_async_copy(b.at[pl.ds(off0, _CH)], ibuf.at[0], sem0).wait()
            pltpu.async_copy(x.at[pl.ds(off1, _CH)], dbuf.at[1], sem1)
            pltpu.async_copy(b.at[pl.ds(off1, _CH)], ibuf.at[1], sem1)
            pltpu.sync_copy(dbuf.at[0], acc.at[ibuf.at[0]], add=True)
            pltpu.make_async_copy(x.at[pl.ds(off1, _CH)], dbuf.at[1], sem1).wait()
            pltpu.make_async_copy(b.at[pl.ds(off1, _CH)], ibuf.at[1], sem1).wait()

            @pl.when(i0 + 2 < _NCH)
            def _():
                off2 = pl.multiple_of(base + (i0 + 2) * _CH, 8)
                pltpu.async_copy(x.at[pl.ds(off2, _CH)], dbuf.at[0], sem0)
                pltpu.async_copy(b.at[pl.ds(off2, _CH)], ibuf.at[0], sem0)

            pltpu.sync_copy(dbuf.at[1], acc.at[ibuf.at[1]], add=True)
            return carry

        lax.fori_loop(0, _NCH // 2, body, 0)

    plsc.subcore_barrier()
    for p in range(3):
        pltpu.sync_copy(accs[p].at[pl.ds(row0, 16)],
                        sums_out.at[p, pl.ds(row0, 16)])


def _hist_tc(b_ref, o_ref):
    p = pl.program_id(0)
    i = pl.program_id(1)

    @pl.when(i == 0)
    def _():
        o_ref[...] = jnp.zeros_like(o_ref)

    ids = b_ref[0]
    e = lax.broadcasted_iota(jnp.int32, (1, _E), 1)
    eq = (ids == e).astype(jnp.float32)
    o_ref[0] += jnp.sum(eq, axis=0, keepdims=True)


def _mlp_tc(s_ref, c_ref, w1_ref, b1_ref, w2_ref, b2_ref, w3_ref, b3_ref, o_ref):
    c = jnp.maximum(c_ref[...], 1.0)
    m = s_ref[...] / c
    h = (jnp.dot(m[0:_E], w1_ref[0:_D], preferred_element_type=jnp.float32)
         + jnp.dot(m[_E:2 * _E], w1_ref[_D:2 * _D], preferred_element_type=jnp.float32)
         + jnp.dot(m[2 * _E:3 * _E], w1_ref[2 * _D:3 * _D], preferred_element_type=jnp.float32))
    h = jax.nn.relu(h + b1_ref[0])
    h = jax.nn.relu(jnp.dot(h, w2_ref[...], preferred_element_type=jnp.float32) + b2_ref[0])
    o_ref[...] = jnp.dot(h, w3_ref[...], preferred_element_type=jnp.float32) + b3_ref[0]


def kernel(x_u, x_v, x_y, batch_u, batch_v, batch_y, W1, b1, W2, b2, W3, b3):
    z_pl = jnp.zeros((_E, _D), jnp.float32)
    sums = _segsum_sc(x_u, x_v, x_y, batch_u, batch_v, batch_y, z_pl)

    ids3 = jnp.stack([batch_u, batch_v, batch_y])
    ids3 = jnp.pad(ids3, ((0, 0), (0, _NPAD - _N)), constant_values=-1)
    ids3 = ids3.reshape(3, _NPAD, 1)
    cnts = pl.pallas_call(
        _hist_tc,
        grid=(3, _NPAD // _HB),
        in_specs=[pl.BlockSpec((1, _HB, 1), lambda p, i: (p, i, 0))],
        out_specs=pl.BlockSpec((1, 1, _E), lambda p, i: (p, 0, 0)),
        out_shape=jax.ShapeDtypeStruct((3, 1, _E), jnp.float32),
    )(ids3)

    s2 = sums.reshape(3 * _E, _D)
    c2 = cnts.reshape(3 * _E, 1)
    W3p = jnp.pad(W3, ((0, 0), (0, 128 - W3.shape[1])))
    b3p = jnp.pad(b3, (0, 128 - b3.shape[0])).reshape(1, 128)
    out = pl.pallas_call(
        _mlp_tc,
        out_shape=jax.ShapeDtypeStruct((_E, 128), jnp.float32),
    )(s2, c2, W1, b1.reshape(1, _D), W2, b2.reshape(1, 64), W3p, b3p)
    return out[:, :3]
